# Initial kernel scaffold; baseline (speedup 1.0000x reference)
import jax
import jax.numpy as jnp
from jax.experimental import pallas as pl


def kernel(x, edge_index, W1_l, b1, W1_r, W2_l, b2, W2_r):
    raise NotImplementedError("write your pallas kernel here")


# speedup vs baseline: 3.0882x; 3.0882x over previous
import functools

import jax
import jax.numpy as jnp
from jax import lax
from jax.experimental import pallas as pl
from jax.experimental.pallas import tpu as pltpu
from jax.experimental.pallas import tpu_sc as plsc

N_NODES = 10000
D = 128
N_EDGES = 320000

NUM_CORES = 2
NUM_SUBCORES = 16
NUM_TILES = NUM_CORES * NUM_SUBCORES

NPAD = 10240
ROWS_PER_TILE = NPAD // NUM_SUBCORES
DUMMY_DST = N_NODES

EPAD = 327680
E_PER_TILE = EPAD // NUM_TILES
CHUNK = 128
N_CHUNKS = E_PER_TILE // CHUNK


def _make_seg_sum(compute_deg: bool):
    mesh = plsc.VectorSubcoreMesh(core_axis_name="c", subcore_axis_name="s")
    out_type = [jax.ShapeDtypeStruct((NUM_CORES, NPAD, D), jnp.float32)]
    if compute_deg:
        out_type.append(jax.ShapeDtypeStruct((NUM_CORES, NPAD), jnp.float32))
    scratch_types = [
        pltpu.VMEM((CHUNK,), jnp.int32),
        pltpu.VMEM((CHUNK,), jnp.int32),
        pltpu.VMEM((CHUNK, D), jnp.float32),
        pltpu.VMEM((CHUNK,), jnp.float32),
        pltpu.VMEM_SHARED((NPAD, D), jnp.float32),
        pltpu.VMEM_SHARED((NPAD,), jnp.float32),
        pltpu.SemaphoreType.DMA,
    ]

    def body(*refs):
        if compute_deg:
            (x_hbm, src_hbm, dst_hbm, z2d_hbm, z1d_hbm, ones_hbm,
             s_out, deg_out,
             src_v, dst_v, rows_v, ones_v, acc_sh, deg_sh, sem) = refs
        else:
            (x_hbm, src_hbm, dst_hbm, z2d_hbm, z1d_hbm, ones_hbm,
             s_out,
             src_v, dst_v, rows_v, ones_v, acc_sh, deg_sh, sem) = refs
        c = lax.axis_index("c")
        s = lax.axis_index("s")
        tid = c * NUM_SUBCORES + s
        rbase = s * ROWS_PER_TILE

        pltpu.sync_copy(z2d_hbm, acc_sh.at[pl.ds(rbase, ROWS_PER_TILE)])
        pltpu.sync_copy(z1d_hbm, deg_sh.at[pl.ds(rbase, ROWS_PER_TILE)])
        pltpu.sync_copy(ones_hbm, ones_v)
        plsc.subcore_barrier()

        ebase = tid * E_PER_TILE

        def step(i, carry):
            off = ebase + i * CHUNK
            pltpu.sync_copy(src_hbm.at[pl.ds(off, CHUNK)], src_v)
            pltpu.sync_copy(dst_hbm.at[pl.ds(off, CHUNK)], dst_v)
            pltpu.async_copy(x_hbm.at[src_v], rows_v, sem).wait()
            pltpu.sync_copy(rows_v, acc_sh.at[dst_v], add=True)
            if compute_deg:
                pltpu.sync_copy(ones_v, deg_sh.at[dst_v], add=True)
            return carry

        lax.fori_loop(0, N_CHUNKS, step, 0)
        plsc.subcore_barrier()

        pltpu.sync_copy(acc_sh.at[pl.ds(rbase, ROWS_PER_TILE)],
                        s_out.at[c, pl.ds(rbase, ROWS_PER_TILE)])
        if compute_deg:
            pltpu.sync_copy(deg_sh.at[pl.ds(rbase, ROWS_PER_TILE)],
                            deg_out.at[c, pl.ds(rbase, ROWS_PER_TILE)])

    return pl.kernel(body, out_type=out_type, scratch_types=scratch_types,
                     mesh=mesh)


_seg_sum_deg = _make_seg_sum(True)
_seg_sum = _make_seg_sum(False)


BM = 1024


def _dense_body(relu, s_ref, degt_ref, x_ref, wl_ref, b_ref, wr_ref, o_ref):
    deg = degt_ref[:, 0:1] + degt_ref[:, 1:2]
    inv = 1.0 / jnp.maximum(deg, 1.0)
    agg = (s_ref[0] + s_ref[1]) * inv
    y = (jnp.dot(agg, wl_ref[...], preferred_element_type=jnp.float32)
         + b_ref[...]
         + jnp.dot(x_ref[...], wr_ref[...], preferred_element_type=jnp.float32))
    o_ref[...] = jnp.maximum(y, 0.0) if relu else y


def _dense(s, degt, x, w_l, b, w_r, relu):
    grid = (NPAD // BM,)
    return pl.pallas_call(
        functools.partial(_dense_body, relu),
        grid=grid,
        in_specs=[
            pl.BlockSpec((NUM_CORES, BM, D), lambda i: (0, i, 0)),
            pl.BlockSpec((BM, NUM_CORES), lambda i: (i, 0)),
            pl.BlockSpec((BM, D), lambda i: (i, 0)),
            pl.BlockSpec((D, D), lambda i: (0, 0)),
            pl.BlockSpec((1, D), lambda i: (0, 0)),
            pl.BlockSpec((D, D), lambda i: (0, 0)),
        ],
        out_specs=pl.BlockSpec((BM, D), lambda i: (i, 0)),
        out_shape=jax.ShapeDtypeStruct((NPAD, D), jnp.float32),
        compiler_params=pltpu.CompilerParams(
            dimension_semantics=("arbitrary",)),
    )(s, degt, x, w_l, b.reshape(1, D), w_r)


def kernel(x, edge_index, W1_l, b1, W1_r, W2_l, b2, W2_r):
    src = edge_index[0].astype(jnp.int32)
    dst = edge_index[1].astype(jnp.int32)
    src = jnp.concatenate([src, jnp.zeros((EPAD - N_EDGES,), jnp.int32)])
    dst = jnp.concatenate([dst, jnp.full((EPAD - N_EDGES,), DUMMY_DST, jnp.int32)])
    x_pad = jnp.pad(x, ((0, NPAD - N_NODES), (0, 0)))

    z2d = jnp.zeros((ROWS_PER_TILE, D), jnp.float32)
    z1d = jnp.zeros((ROWS_PER_TILE,), jnp.float32)
    ones = jnp.ones((CHUNK,), jnp.float32)

    s1, degp = _seg_sum_deg(x_pad, src, dst, z2d, z1d, ones)
    degt = degp.T
    h = _dense(s1, degt, x_pad, W1_l, b1, W1_r, relu=True)
    (s2,) = _seg_sum(h, src, dst, z2d, z1d, ones)
    out = _dense(s2, degt, h, W2_l, b2, W2_r, relu=False)
    return out[:N_NODES]


# speedup vs baseline: 3.2129x; 1.0404x over previous
import functools

import jax
import jax.numpy as jnp
from jax import lax
from jax.experimental import pallas as pl
from jax.experimental.pallas import tpu as pltpu
from jax.experimental.pallas import tpu_sc as plsc

N_NODES = 10000
D = 128
N_EDGES = 320000

NUM_CORES = 2
NUM_SUBCORES = 16
NUM_TILES = NUM_CORES * NUM_SUBCORES

NPAD = 10240
ROWS_PER_TILE = NPAD // NUM_SUBCORES
DUMMY_DST = N_NODES

EPAD = 327680
E_PER_TILE = EPAD // NUM_TILES
CHUNK = 128
N_CHUNKS = E_PER_TILE // CHUNK
NBUF = 2
N_GROUPS = N_CHUNKS // NBUF
N_ITERS = N_GROUPS // 2


def _seg_body(x_hbm, comb_hbm, z2d_hbm, s_out,
              stg0, stg1, r0, r1, acc_sh, isem, gsem, ssem):
    rows = (r0, r1)
    c = lax.axis_index("c")
    s = lax.axis_index("s")
    tid = c * NUM_SUBCORES + s
    rbase = s * ROWS_PER_TILE
    gb = tid * N_GROUPS

    pltpu.sync_copy(z2d_hbm, acc_sh.at[pl.ds(rbase, ROWS_PER_TILE)])
    pltpu.sync_copy(comb_hbm.at[gb], stg0)
    plsc.subcore_barrier()

    def it(m, carry):
        g1 = gb + 2 * m + 1

        @pl.when(m > 0)
        def _():
            for b in range(NBUF):
                pltpu.make_async_copy(rows[b], acc_sh.at[stg1.at[1, b]],
                                      ssem.at[b]).wait()
        pltpu.async_copy(comb_hbm.at[g1], stg1, isem)
        for b in range(NBUF):
            pltpu.async_copy(x_hbm.at[stg0.at[0, b]], rows[b], gsem.at[b])
        for b in range(NBUF):
            pltpu.make_async_copy(x_hbm.at[stg0.at[0, b]], rows[b],
                                  gsem.at[b]).wait()
            pltpu.async_copy(rows[b], acc_sh.at[stg0.at[1, b]], ssem.at[b],
                             add=True)
        pltpu.make_async_copy(comb_hbm.at[g1], stg1, isem).wait()

        for b in range(NBUF):
            pltpu.make_async_copy(rows[b], acc_sh.at[stg0.at[1, b]],
                                  ssem.at[b]).wait()
        @pl.when(m < N_ITERS - 1)
        def _():
            pltpu.async_copy(comb_hbm.at[g1 + 1], stg0, isem)
        for b in range(NBUF):
            pltpu.async_copy(x_hbm.at[stg1.at[0, b]], rows[b], gsem.at[b])
        for b in range(NBUF):
            pltpu.make_async_copy(x_hbm.at[stg1.at[0, b]], rows[b],
                                  gsem.at[b]).wait()
            pltpu.async_copy(rows[b], acc_sh.at[stg1.at[1, b]], ssem.at[b],
                             add=True)
        @pl.when(m < N_ITERS - 1)
        def _():
            pltpu.make_async_copy(comb_hbm.at[g1 + 1], stg0, isem).wait()
        return carry

    lax.fori_loop(0, N_ITERS, it, 0)
    for b in range(NBUF):
        pltpu.make_async_copy(rows[b], acc_sh.at[stg1.at[1, b]],
                              ssem.at[b]).wait()
    plsc.subcore_barrier()

    pltpu.sync_copy(acc_sh.at[pl.ds(rbase, ROWS_PER_TILE)],
                    s_out.at[c, pl.ds(rbase, ROWS_PER_TILE)])


_seg_sum = pl.kernel(
    _seg_body,
    out_type=[jax.ShapeDtypeStruct((NUM_CORES, NPAD, D), jnp.float32)],
    scratch_types=[
        pltpu.VMEM((2, NBUF, CHUNK), jnp.int32),
        pltpu.VMEM((2, NBUF, CHUNK), jnp.int32),
        pltpu.VMEM((CHUNK, D), jnp.float32),
        pltpu.VMEM((CHUNK, D), jnp.float32),
        pltpu.VMEM_SHARED((NPAD, D), jnp.float32),
        pltpu.SemaphoreType.DMA,
        pltpu.SemaphoreType.DMA((NBUF,)),
        pltpu.SemaphoreType.DMA((NBUF,)),
    ],
    mesh=plsc.VectorSubcoreMesh(core_axis_name="c", subcore_axis_name="s"),
)


def _deg_body(dst_hbm, z1d_hbm, ones_hbm, deg_out,
              dst_idx, ones_v, deg_sh, dsem):
    c = lax.axis_index("c")
    s = lax.axis_index("s")
    tid = c * NUM_SUBCORES + s
    rbase = s * ROWS_PER_TILE
    pltpu.sync_copy(z1d_hbm, deg_sh.at[pl.ds(rbase, ROWS_PER_TILE)])
    pltpu.sync_copy(ones_hbm, ones_v)
    pltpu.sync_copy(dst_hbm.at[pl.ds(tid * N_CHUNKS, N_CHUNKS)], dst_idx)
    plsc.subcore_barrier()

    def fire(j, carry):
        pltpu.async_copy(ones_v, deg_sh.at[dst_idx.at[j]], dsem, add=True)
        return carry
    lax.fori_loop(0, N_CHUNKS, fire, 0)

    def drain(j, carry):
        pltpu.make_async_copy(ones_v, deg_sh.at[dst_idx.at[0]], dsem).wait()
        return carry
    lax.fori_loop(0, N_CHUNKS, drain, 0)
    plsc.subcore_barrier()
    pltpu.sync_copy(deg_sh.at[pl.ds(rbase, ROWS_PER_TILE)],
                    deg_out.at[c, pl.ds(rbase, ROWS_PER_TILE)])


_deg_sum = pl.kernel(
    _deg_body,
    out_type=[jax.ShapeDtypeStruct((NUM_CORES, NPAD), jnp.float32)],
    scratch_types=[
        pltpu.VMEM((N_CHUNKS, CHUNK), jnp.int32),
        pltpu.VMEM((CHUNK,), jnp.float32),
        pltpu.VMEM_SHARED((NPAD,), jnp.float32),
        pltpu.SemaphoreType.DMA,
    ],
    mesh=plsc.VectorSubcoreMesh(core_axis_name="c", subcore_axis_name="s"),
)


BM = 1024


def _dense_body(relu, s_ref, degt_ref, x_ref, wl_ref, b_ref, wr_ref, o_ref):
    deg = degt_ref[:, 0:1] + degt_ref[:, 1:2]
    inv = 1.0 / jnp.maximum(deg, 1.0)
    agg = (s_ref[0] + s_ref[1]) * inv
    y = (jnp.dot(agg, wl_ref[...], preferred_element_type=jnp.float32)
         + b_ref[...]
         + jnp.dot(x_ref[...], wr_ref[...], preferred_element_type=jnp.float32))
    o_ref[...] = jnp.maximum(y, 0.0) if relu else y


def _dense(s, degt, x, w_l, b, w_r, relu):
    grid = (NPAD // BM,)
    return pl.pallas_call(
        functools.partial(_dense_body, relu),
        grid=grid,
        in_specs=[
            pl.BlockSpec((NUM_CORES, BM, D), lambda i: (0, i, 0)),
            pl.BlockSpec((BM, NUM_CORES), lambda i: (i, 0)),
            pl.BlockSpec((BM, D), lambda i: (i, 0)),
            pl.BlockSpec((D, D), lambda i: (0, 0)),
            pl.BlockSpec((1, D), lambda i: (0, 0)),
            pl.BlockSpec((D, D), lambda i: (0, 0)),
        ],
        out_specs=pl.BlockSpec((BM, D), lambda i: (i, 0)),
        out_shape=jax.ShapeDtypeStruct((NPAD, D), jnp.float32),
        compiler_params=pltpu.CompilerParams(
            dimension_semantics=("arbitrary",)),
    )(s, degt, x, w_l, b.reshape(1, D), w_r)


def kernel(x, edge_index, W1_l, b1, W1_r, W2_l, b2, W2_r):
    src = edge_index[0].astype(jnp.int32)
    dst = edge_index[1].astype(jnp.int32)
    src = jnp.concatenate([src, jnp.zeros((EPAD - N_EDGES,), jnp.int32)])
    dst = jnp.concatenate([dst, jnp.full((EPAD - N_EDGES,), DUMMY_DST, jnp.int32)])
    src4 = src.reshape(NUM_TILES, N_GROUPS, NBUF, CHUNK)
    dst4 = dst.reshape(NUM_TILES, N_GROUPS, NBUF, CHUNK)
    comb = jnp.stack([src4, dst4], axis=2).reshape(
        NUM_TILES * N_GROUPS, 2, NBUF, CHUNK)
    dst2d = dst.reshape(NUM_TILES * N_CHUNKS, CHUNK)
    x_pad = jnp.pad(x, ((0, NPAD - N_NODES), (0, 0)))

    z2d = jnp.zeros((ROWS_PER_TILE, D), jnp.float32)
    z1d = jnp.zeros((ROWS_PER_TILE,), jnp.float32)
    ones = jnp.ones((CHUNK,), jnp.float32)

    (degp,) = _deg_sum(dst2d, z1d, ones)
    (s1,) = _seg_sum(x_pad, comb, z2d)
    degt = degp.T
    h = _dense(s1, degt, x_pad, W1_l, b1, W1_r, relu=True)
    (s2,) = _seg_sum(h, comb, z2d)
    out = _dense(s2, degt, h, W2_l, b2, W2_r, relu=False)
    return out[:N_NODES]


# speedup vs baseline: 5.3433x; 1.6631x over previous
import functools

import jax
import jax.numpy as jnp
from jax import lax
from jax.experimental import pallas as pl
from jax.experimental.pallas import tpu as pltpu
from jax.experimental.pallas import tpu_sc as plsc

N_NODES = 10000
D = 128
N_EDGES = 320000

NUM_CORES = 2
NUM_SUBCORES = 16
NUM_TILES = NUM_CORES * NUM_SUBCORES

NPAD = 10112
ROWS_PER_TILE = NPAD // NUM_SUBCORES
DUMMY_DST = N_NODES

EPAD = 327680
E_PER_TILE = EPAD // NUM_TILES
CHUNK = 128
N_CHUNKS = E_PER_TILE // CHUNK
NBUF = 2
N_GROUPS = N_CHUNKS // NBUF
N_ITERS = N_GROUPS // 2


def _seg_body(x_hbm, comb_hbm, z2d_hbm, s_out,
              stg0, stg1, rb0, rb1, rf, acc_sh, isem, gsem, ssem):
    rbf = (rb0, rb1)
    c = lax.axis_index("c")
    s = lax.axis_index("s")
    tid = c * NUM_SUBCORES + s
    rbase = s * ROWS_PER_TILE
    gb = tid * N_GROUPS

    pltpu.sync_copy(z2d_hbm, acc_sh.at[pl.ds(rbase, ROWS_PER_TILE)])
    pltpu.sync_copy(comb_hbm.at[gb], stg0)
    plsc.subcore_barrier()

    def widen(src_bf, dst_f32):
        def row(r, carry):
            for cc in range(D // 32):
                v = src_bf[r, pl.ds(cc * 32, 32)]
                lo, hi = plsc.unpack(v, format=plsc.PackFormat.INTERLEAVED)
                dst_f32[r, pl.ds(cc * 16, 16)] = lo
                dst_f32[r, pl.ds(D // 2 + cc * 16, 16)] = hi
            return carry
        lax.fori_loop(0, CHUNK, row, 0)

    def wait_gather(src_idx_row, b):
        pltpu.make_async_copy(x_hbm.at[src_idx_row], rbf[b], gsem.at[b]).wait()

    def wait_scatter():
        pltpu.make_async_copy(rf, acc_sh.at[stg0.at[1, 0]], ssem).wait()

    for b in range(NBUF):
        pltpu.async_copy(x_hbm.at[stg0.at[0, b]], rbf[b], gsem.at[b])

    def it(m, carry):
        g0 = gb + 2 * m
        not_last = m < N_ITERS - 1

        wait_gather(stg0.at[0, 0], 0)
        @pl.when(m > 0)
        def _():
            wait_scatter()
        pltpu.async_copy(comb_hbm.at[g0 + 1], stg1, isem)
        widen(rbf[0], rf)
        pltpu.async_copy(rf, acc_sh.at[stg0.at[1, 0]], ssem, add=True)
        pltpu.make_async_copy(comb_hbm.at[g0 + 1], stg1, isem).wait()
        pltpu.async_copy(x_hbm.at[stg1.at[0, 0]], rbf[0], gsem.at[0])

        wait_gather(stg0.at[0, 1], 1)
        wait_scatter()
        widen(rbf[1], rf)
        pltpu.async_copy(rf, acc_sh.at[stg0.at[1, 1]], ssem, add=True)
        pltpu.async_copy(x_hbm.at[stg1.at[0, 1]], rbf[1], gsem.at[1])

        wait_gather(stg1.at[0, 0], 0)
        wait_scatter()
        @pl.when(not_last)
        def _():
            pltpu.async_copy(comb_hbm.at[g0 + 2], stg0, isem)
        widen(rbf[0], rf)
        pltpu.async_copy(rf, acc_sh.at[stg1.at[1, 0]], ssem, add=True)
        @pl.when(not_last)
        def _():
            pltpu.make_async_copy(comb_hbm.at[g0 + 2], stg0, isem).wait()
            pltpu.async_copy(x_hbm.at[stg0.at[0, 0]], rbf[0], gsem.at[0])

        wait_gather(stg1.at[0, 1], 1)
        wait_scatter()
        widen(rbf[1], rf)
        pltpu.async_copy(rf, acc_sh.at[stg1.at[1, 1]], ssem, add=True)
        @pl.when(not_last)
        def _():
            pltpu.async_copy(x_hbm.at[stg0.at[0, 1]], rbf[1], gsem.at[1])
        return carry

    lax.fori_loop(0, N_ITERS, it, 0)
    wait_scatter()
    plsc.subcore_barrier()

    pltpu.sync_copy(acc_sh.at[pl.ds(rbase, ROWS_PER_TILE)],
                    s_out.at[c, pl.ds(rbase, ROWS_PER_TILE)])


_seg_sum = pl.kernel(
    _seg_body,
    out_type=[jax.ShapeDtypeStruct((NUM_CORES, NPAD, D), jnp.float32)],
    scratch_types=[
        pltpu.VMEM((2, NBUF, CHUNK), jnp.int32),
        pltpu.VMEM((2, NBUF, CHUNK), jnp.int32),
        pltpu.VMEM((CHUNK, D), jnp.bfloat16),
        pltpu.VMEM((CHUNK, D), jnp.bfloat16),
        pltpu.VMEM((CHUNK, D), jnp.float32),
        pltpu.VMEM_SHARED((NPAD, D), jnp.float32),
        pltpu.SemaphoreType.DMA,
        pltpu.SemaphoreType.DMA((NBUF,)),
        pltpu.SemaphoreType.DMA,
    ],
    mesh=plsc.VectorSubcoreMesh(core_axis_name="c", subcore_axis_name="s"),
    compiler_params=pltpu.CompilerParams(use_tc_tiling_on_sc=False,
                                         needs_layout_passes=False),
)


def _deg_body(dst_hbm, z1d_hbm, ones_hbm, deg_out,
              dst_idx, ones_v, deg_sh, dsem):
    c = lax.axis_index("c")
    s = lax.axis_index("s")
    tid = c * NUM_SUBCORES + s
    rbase = s * ROWS_PER_TILE
    pltpu.sync_copy(z1d_hbm, deg_sh.at[pl.ds(rbase, ROWS_PER_TILE)])
    pltpu.sync_copy(ones_hbm, ones_v)
    pltpu.sync_copy(dst_hbm.at[pl.ds(tid * N_CHUNKS, N_CHUNKS)], dst_idx)
    plsc.subcore_barrier()

    def fire(j, carry):
        pltpu.async_copy(ones_v, deg_sh.at[dst_idx.at[j]], dsem, add=True)
        return carry
    lax.fori_loop(0, N_CHUNKS, fire, 0)

    def drain(j, carry):
        pltpu.make_async_copy(ones_v, deg_sh.at[dst_idx.at[0]], dsem).wait()
        return carry
    lax.fori_loop(0, N_CHUNKS, drain, 0)
    plsc.subcore_barrier()
    pltpu.sync_copy(deg_sh.at[pl.ds(rbase, ROWS_PER_TILE)],
                    deg_out.at[c, pl.ds(rbase, ROWS_PER_TILE)])


_deg_sum = pl.kernel(
    _deg_body,
    out_type=[jax.ShapeDtypeStruct((NUM_CORES, NPAD), jnp.float32)],
    scratch_types=[
        pltpu.VMEM((N_CHUNKS, CHUNK), jnp.int32),
        pltpu.VMEM((CHUNK,), jnp.float32),
        pltpu.VMEM_SHARED((NPAD,), jnp.float32),
        pltpu.SemaphoreType.DMA,
    ],
    mesh=plsc.VectorSubcoreMesh(core_axis_name="c", subcore_axis_name="s"),
    compiler_params=pltpu.CompilerParams(use_tc_tiling_on_sc=False),
)


BM = 1264


def _dense_body(relu, s_ref, degt_ref, x_ref, wl_ref, b_ref, wr_ref, o_ref):
    deg = degt_ref[:, 0:1] + degt_ref[:, 1:2]
    inv = 1.0 / jnp.maximum(deg, 1.0)
    agg = (s_ref[0] + s_ref[1]) * inv
    y = (jnp.dot(agg, wl_ref[...], preferred_element_type=jnp.float32)
         + b_ref[...]
         + jnp.dot(x_ref[...], wr_ref[...], preferred_element_type=jnp.float32))
    o_ref[...] = jnp.maximum(y, 0.0) if relu else y


def _dense(s, degt, x, w_l, b, w_r, relu):
    grid = (NPAD // BM,)
    return pl.pallas_call(
        functools.partial(_dense_body, relu),
        grid=grid,
        in_specs=[
            pl.BlockSpec((NUM_CORES, BM, D), lambda i: (0, i, 0)),
            pl.BlockSpec((BM, NUM_CORES), lambda i: (i, 0)),
            pl.BlockSpec((BM, D), lambda i: (i, 0)),
            pl.BlockSpec((D, D), lambda i: (0, 0)),
            pl.BlockSpec((1, D), lambda i: (0, 0)),
            pl.BlockSpec((D, D), lambda i: (0, 0)),
        ],
        out_specs=pl.BlockSpec((BM, D), lambda i: (i, 0)),
        out_shape=jax.ShapeDtypeStruct((NPAD, D), jnp.float32),
        compiler_params=pltpu.CompilerParams(
            dimension_semantics=("arbitrary",)),
    )(s, degt, x, w_l, b.reshape(1, D), w_r)


def _interleave_bf16(a):
    return jnp.stack([a[:, :D // 2], a[:, D // 2:]],
                     axis=-1).reshape(a.shape[0], D).astype(jnp.bfloat16)


def kernel(x, edge_index, W1_l, b1, W1_r, W2_l, b2, W2_r):
    src = edge_index[0].astype(jnp.int32)
    dst = edge_index[1].astype(jnp.int32)
    src = jnp.concatenate([src, jnp.zeros((EPAD - N_EDGES,), jnp.int32)])
    dst = jnp.concatenate([dst, jnp.full((EPAD - N_EDGES,), DUMMY_DST, jnp.int32)])
    src4 = src.reshape(NUM_TILES, N_GROUPS, NBUF, CHUNK)
    dst4 = dst.reshape(NUM_TILES, N_GROUPS, NBUF, CHUNK)
    comb = jnp.stack([src4, dst4], axis=2).reshape(
        NUM_TILES * N_GROUPS, 2, NBUF, CHUNK)
    dst2d = dst.reshape(NUM_TILES * N_CHUNKS, CHUNK)
    x_pad = jnp.pad(x, ((0, NPAD - N_NODES), (0, 0)))

    z2d = jnp.zeros((ROWS_PER_TILE, D), jnp.float32)
    z1d = jnp.zeros((ROWS_PER_TILE,), jnp.float32)
    ones = jnp.ones((CHUNK,), jnp.float32)

    (degp,) = _deg_sum(dst2d, z1d, ones)
    (s1,) = _seg_sum(_interleave_bf16(x_pad), comb, z2d)
    degt = degp.T
    h = _dense(s1, degt, x_pad, W1_l, b1, W1_r, relu=True)
    (s2,) = _seg_sum(_interleave_bf16(h), comb, z2d)
    out = _dense(s2, degt, h, W2_l, b2, W2_r, relu=False)
    return out[:N_NODES]


# speedup vs baseline: 6.1286x; 1.1470x over previous
import functools

import numpy as np

import jax
import jax.numpy as jnp
from jax import lax
from jax.experimental import pallas as pl
from jax.experimental.pallas import tpu as pltpu
from jax.experimental.pallas import tpu_sc as plsc

N_NODES = 10000
D = 128
N_EDGES = 320000

NUM_CORES = 2
NUM_SUBCORES = 16
NUM_TILES = NUM_CORES * NUM_SUBCORES

NPAD = 10112
ROWS_PER_TILE = NPAD // NUM_SUBCORES
DUMMY_DST = N_NODES

EPAD = 327680
E_PER_TILE = EPAD // NUM_TILES
CHUNK = 128
N_CHUNKS = E_PER_TILE // CHUNK
NBUF = 2
N_GROUPS = N_CHUNKS // NBUF
N_ITERS = N_GROUPS // 2

UNPACK_PERM = np.concatenate([np.arange(0, D, 2), np.arange(1, D, 2)])


def _make_seg_body(compute_deg):
    def body(*refs):
        if compute_deg:
            (x_hbm, comb_hbm, z2d_hbm, z1d_hbm, ones_hbm, s_out, deg_out,
             stg0, stg1, rb0, rb1, rf, ones_v, acc_sh, deg_sh,
             isem, gsem, ssem, dsem) = refs
        else:
            (x_hbm, comb_hbm, z2d_hbm, s_out,
             stg0, stg1, rb0, rb1, rf, acc_sh, isem, gsem, ssem) = refs
        rbf = (rb0, rb1)
        c = lax.axis_index("c")
        s = lax.axis_index("s")
        tid = c * NUM_SUBCORES + s
        rbase = s * ROWS_PER_TILE
        gb = tid * N_GROUPS

        pltpu.sync_copy(z2d_hbm, acc_sh.at[pl.ds(rbase, ROWS_PER_TILE)])
        if compute_deg:
            pltpu.sync_copy(z1d_hbm, deg_sh.at[pl.ds(rbase, ROWS_PER_TILE)])
            pltpu.sync_copy(ones_hbm, ones_v)
        pltpu.sync_copy(comb_hbm.at[gb], stg0)
        plsc.subcore_barrier()

        def widen(src_bf, dst_f32):
            def row(r, carry):
                for cc in range(D // 32):
                    v = src_bf[r, pl.ds(cc * 32, 32)]
                    lo, hi = plsc.unpack(
                        v, format=plsc.PackFormat.INTERLEAVED)
                    dst_f32[r, pl.ds(cc * 16, 16)] = lo
                    dst_f32[r, pl.ds(D // 2 + cc * 16, 16)] = hi
                return carry
            lax.fori_loop(0, CHUNK, row, 0)

        def wait_gather(src_idx_row, b):
            pltpu.make_async_copy(x_hbm.at[src_idx_row], rbf[b],
                                  gsem.at[b]).wait()

        def wait_scatter():
            pltpu.make_async_copy(rf, acc_sh.at[stg0.at[1, 0]], ssem).wait()

        def fire_deg(idx_row):
            if compute_deg:
                pltpu.async_copy(ones_v, deg_sh.at[idx_row], dsem, add=True)

        def wait_deg2():
            if compute_deg:
                for _ in range(2):
                    pltpu.make_async_copy(ones_v, deg_sh.at[stg0.at[1, 0]],
                                          dsem).wait()

        for b in range(NBUF):
            pltpu.async_copy(x_hbm.at[stg0.at[0, b]], rbf[b], gsem.at[b])

        def it(m, carry):
            g0 = gb + 2 * m
            not_last = m < N_ITERS - 1

            wait_gather(stg0.at[0, 0], 0)
            @pl.when(m > 0)
            def _():
                wait_scatter()
                wait_deg2()
            pltpu.async_copy(comb_hbm.at[g0 + 1], stg1, isem)
            widen(rbf[0], rf)
            pltpu.async_copy(rf, acc_sh.at[stg0.at[1, 0]], ssem, add=True)
            fire_deg(stg0.at[1, 0])
            pltpu.make_async_copy(comb_hbm.at[g0 + 1], stg1, isem).wait()
            pltpu.async_copy(x_hbm.at[stg1.at[0, 0]], rbf[0], gsem.at[0])

            wait_gather(stg0.at[0, 1], 1)
            wait_scatter()
            widen(rbf[1], rf)
            pltpu.async_copy(rf, acc_sh.at[stg0.at[1, 1]], ssem, add=True)
            fire_deg(stg0.at[1, 1])
            pltpu.async_copy(x_hbm.at[stg1.at[0, 1]], rbf[1], gsem.at[1])

            wait_gather(stg1.at[0, 0], 0)
            wait_scatter()
            wait_deg2()
            @pl.when(not_last)
            def _():
                pltpu.async_copy(comb_hbm.at[g0 + 2], stg0, isem)
            widen(rbf[0], rf)
            pltpu.async_copy(rf, acc_sh.at[stg1.at[1, 0]], ssem, add=True)
            fire_deg(stg1.at[1, 0])
            @pl.when(not_last)
            def _():
                pltpu.make_async_copy(comb_hbm.at[g0 + 2], stg0, isem).wait()
                pltpu.async_copy(x_hbm.at[stg0.at[0, 0]], rbf[0], gsem.at[0])

            wait_gather(stg1.at[0, 1], 1)
            wait_scatter()
            widen(rbf[1], rf)
            pltpu.async_copy(rf, acc_sh.at[stg1.at[1, 1]], ssem, add=True)
            fire_deg(stg1.at[1, 1])
            @pl.when(not_last)
            def _():
                pltpu.async_copy(x_hbm.at[stg0.at[0, 1]], rbf[1], gsem.at[1])
            return carry

        lax.fori_loop(0, N_ITERS, it, 0)
        wait_scatter()
        wait_deg2()
        plsc.subcore_barrier()

        pltpu.sync_copy(acc_sh.at[pl.ds(rbase, ROWS_PER_TILE)],
                        s_out.at[c, pl.ds(rbase, ROWS_PER_TILE)])
        if compute_deg:
            pltpu.sync_copy(deg_sh.at[pl.ds(rbase, ROWS_PER_TILE)],
                            deg_out.at[c, pl.ds(rbase, ROWS_PER_TILE)])

    return body


def _make_seg(compute_deg):
    out_type = [jax.ShapeDtypeStruct((NUM_CORES, NPAD, D), jnp.float32)]
    scratch = [
        pltpu.VMEM((2, NBUF, CHUNK), jnp.int32),
        pltpu.VMEM((2, NBUF, CHUNK), jnp.int32),
        pltpu.VMEM((CHUNK, D), jnp.bfloat16),
        pltpu.VMEM((CHUNK, D), jnp.bfloat16),
        pltpu.VMEM((CHUNK, D), jnp.float32),
    ]
    if compute_deg:
        out_type.append(jax.ShapeDtypeStruct((NUM_CORES, NPAD), jnp.float32))
        scratch.append(pltpu.VMEM((CHUNK,), jnp.float32))
    scratch.append(pltpu.VMEM_SHARED((NPAD, D), jnp.float32))
    if compute_deg:
        scratch.append(pltpu.VMEM_SHARED((NPAD,), jnp.float32))
    scratch += [
        pltpu.SemaphoreType.DMA,
        pltpu.SemaphoreType.DMA((NBUF,)),
        pltpu.SemaphoreType.DMA,
    ]
    if compute_deg:
        scratch.append(pltpu.SemaphoreType.DMA)
    return pl.kernel(
        _make_seg_body(compute_deg),
        out_type=out_type,
        scratch_types=scratch,
        mesh=plsc.VectorSubcoreMesh(core_axis_name="c", subcore_axis_name="s"),
        compiler_params=pltpu.CompilerParams(use_tc_tiling_on_sc=False,
                                             needs_layout_passes=False),
    )


_seg_sum_deg = _make_seg(True)
_seg_sum = _make_seg(False)


def _dense_body(relu, s_ref, degt_ref, x_ref, wl_ref, b_ref, wr_ref, *outs):
    deg = degt_ref[:, 0:1] + degt_ref[:, 1:2]
    inv = 1.0 / jnp.maximum(deg, 1.0)
    agg = (s_ref[0] + s_ref[1]) * inv
    y = (jnp.dot(agg, wl_ref[...], preferred_element_type=jnp.float32)
         + b_ref[...]
         + jnp.dot(x_ref[...], wr_ref[...], preferred_element_type=jnp.float32))
    if relu:
        y = jnp.maximum(y, 0.0)
    outs[0][...] = y
    if len(outs) > 1:
        outs[1][...] = y.astype(jnp.bfloat16)


def _dense(s, degt, x, w_l, b, w_r, relu, bf16_out, bm, rows):
    grid = (rows // bm,)
    out_shape = [jax.ShapeDtypeStruct((rows, D), jnp.float32)]
    out_specs = [pl.BlockSpec((bm, D), lambda i: (i, 0))]
    if bf16_out:
        out_shape.append(jax.ShapeDtypeStruct((rows, D), jnp.bfloat16))
        out_specs.append(pl.BlockSpec((bm, D), lambda i: (i, 0)))
    return pl.pallas_call(
        functools.partial(_dense_body, relu),
        grid=grid,
        in_specs=[
            pl.BlockSpec((NUM_CORES, bm, D), lambda i: (0, i, 0)),
            pl.BlockSpec((bm, NUM_CORES), lambda i: (i, 0)),
            pl.BlockSpec((bm, D), lambda i: (i, 0)),
            pl.BlockSpec((D, D), lambda i: (0, 0)),
            pl.BlockSpec((1, D), lambda i: (0, 0)),
            pl.BlockSpec((D, D), lambda i: (0, 0)),
        ],
        out_specs=out_specs,
        out_shape=out_shape,
        compiler_params=pltpu.CompilerParams(
            dimension_semantics=("arbitrary",)),
    )(s, degt, x, w_l, b.reshape(1, D), w_r)


def kernel(x, edge_index, W1_l, b1, W1_r, W2_l, b2, W2_r):
    src = edge_index[0].astype(jnp.int32)
    dst = edge_index[1].astype(jnp.int32)
    src = jnp.concatenate([src, jnp.zeros((EPAD - N_EDGES,), jnp.int32)])
    dst = jnp.concatenate([dst, jnp.full((EPAD - N_EDGES,), DUMMY_DST, jnp.int32)])
    src4 = src.reshape(NUM_TILES, N_GROUPS, NBUF, CHUNK)
    dst4 = dst.reshape(NUM_TILES, N_GROUPS, NBUF, CHUNK)
    comb = jnp.stack([src4, dst4], axis=2).reshape(
        NUM_TILES * N_GROUPS, 2, NBUF, CHUNK)
    x_pad = jnp.pad(x, ((0, NPAD - N_NODES), (0, 0)))

    z2d = jnp.zeros((ROWS_PER_TILE, D), jnp.float32)
    z1d = jnp.zeros((ROWS_PER_TILE,), jnp.float32)
    ones = jnp.ones((CHUNK,), jnp.float32)

    s1, degp = _seg_sum_deg(x_pad.astype(jnp.bfloat16), comb, z2d, z1d, ones)
    degt = degp.T
    h, h_bf = _dense(s1, degt, x_pad, W1_l[UNPACK_PERM], b1, W1_r,
                     relu=True, bf16_out=True, bm=1264, rows=NPAD)
    (s2,) = _seg_sum(h_bf, comb, z2d)
    out, = _dense(s2, degt, h, W2_l[UNPACK_PERM], b2, W2_r,
                  relu=False, bf16_out=False, bm=2000, rows=N_NODES)
    return out


# speedup vs baseline: 9.3090x; 1.5190x over previous
import functools

import jax
import jax.numpy as jnp
from jax import lax
from jax.experimental import pallas as pl
from jax.experimental.pallas import tpu as pltpu
from jax.experimental.pallas import tpu_sc as plsc

N_NODES = 10000
D = 128
DH = D // 2
N_EDGES = 320000

NUM_CORES = 2
NUM_SUBCORES = 16
NUM_TILES = NUM_CORES * NUM_SUBCORES

NPAD = 10112
ROWS_PER_TILE = NPAD // NUM_SUBCORES
DUMMY_DST = N_NODES

EPAD = 327680
E_PER_TILE = EPAD // NUM_TILES
CHUNK = 128
N_CHUNKS = E_PER_TILE // CHUNK
NBUF = 3
N_STEPS = N_CHUNKS + 1
N_ITERS = N_STEPS // NBUF


def _make_seg_body(compute_deg):
    def body(*refs):
        if compute_deg:
            (x_hbm, src_hbm, dst_hbm, z2d_hbm, z1d_hbm, ones_hbm,
             s_out, deg_out,
             src_idx, dst_idx, rv0, rv1, rv2, ones_v,
             xsp_sh, acc_sh, deg_sh, gsem, ssem, dsem) = refs
        else:
            (x_hbm, src_hbm, dst_hbm, z2d_hbm, s_out,
             src_idx, dst_idx, rv0, rv1, rv2,
             xsp_sh, acc_sh, gsem, ssem) = refs
        rv = (rv0, rv1, rv2)
        c = lax.axis_index("c")
        s = lax.axis_index("s")
        tid = c * NUM_SUBCORES + s
        rbase = s * ROWS_PER_TILE

        pltpu.sync_copy(src_hbm.at[pl.ds(tid * N_CHUNKS, N_CHUNKS)], src_idx)
        pltpu.sync_copy(dst_hbm.at[pl.ds(tid * N_CHUNKS, N_CHUNKS)], dst_idx)
        if compute_deg:
            pltpu.sync_copy(z1d_hbm, deg_sh.at[pl.ds(rbase, ROWS_PER_TILE)])
            pltpu.sync_copy(ones_hbm, ones_v)

        for p in range(2):
            deg_pass = compute_deg and p == 0
            pltpu.sync_copy(x_hbm.at[p, pl.ds(rbase, ROWS_PER_TILE)],
                            xsp_sh.at[pl.ds(rbase, ROWS_PER_TILE)])
            pltpu.sync_copy(z2d_hbm, acc_sh.at[pl.ds(rbase, ROWS_PER_TILE)])
            plsc.subcore_barrier()

            def step(k, u):
                j = k * NBUF + u
                b = rv[u]
                up = (u - 1) % NBUF

                @pl.when(k > 0)
                def _():
                    pltpu.make_async_copy(b, acc_sh.at[dst_idx.at[0]],
                                          ssem.at[u]).wait()
                @pl.when(j < N_CHUNKS)
                def _():
                    pltpu.async_copy(xsp_sh.at[src_idx.at[j]], b, gsem.at[u])

                i = j - 1
                bp = rv[up]
                @pl.when(i >= 0)
                def _():
                    pltpu.make_async_copy(xsp_sh.at[src_idx.at[i]], bp,
                                          gsem.at[up]).wait()
                    pltpu.async_copy(bp, acc_sh.at[dst_idx.at[i]],
                                     ssem.at[up], add=True)
                    if deg_pass:
                        pltpu.async_copy(ones_v, deg_sh.at[dst_idx.at[i]],
                                         dsem, add=True)

            def it(k, carry):
                for u in range(NBUF):
                    step(k, u)
                return carry

            lax.fori_loop(0, N_ITERS, it, 0)
            for u in range(NBUF):
                if sum(1 for i in range(N_CHUNKS) if i % NBUF == u) > N_ITERS - 1:
                    pltpu.make_async_copy(rv[u], acc_sh.at[dst_idx.at[0]],
                                          ssem.at[u]).wait()
            if deg_pass:
                def dwait(i, carry):
                    pltpu.make_async_copy(ones_v, deg_sh.at[dst_idx.at[0]],
                                          dsem).wait()
                    return carry
                lax.fori_loop(0, N_CHUNKS, dwait, 0)
            plsc.subcore_barrier()

            pltpu.sync_copy(acc_sh.at[pl.ds(rbase, ROWS_PER_TILE)],
                            s_out.at[c, p, pl.ds(rbase, ROWS_PER_TILE)])
            if deg_pass:
                pltpu.sync_copy(deg_sh.at[pl.ds(rbase, ROWS_PER_TILE)],
                                deg_out.at[c, pl.ds(rbase, ROWS_PER_TILE)])
            plsc.subcore_barrier()

    return body


def _make_seg(compute_deg):
    out_type = [jax.ShapeDtypeStruct((NUM_CORES, 2, NPAD, DH), jnp.float32)]
    if compute_deg:
        out_type.append(jax.ShapeDtypeStruct((NUM_CORES, NPAD), jnp.float32))
    scratch = [
        pltpu.VMEM((N_CHUNKS, CHUNK), jnp.int32),
        pltpu.VMEM((N_CHUNKS, CHUNK), jnp.int32),
        pltpu.VMEM((CHUNK, DH), jnp.float32),
        pltpu.VMEM((CHUNK, DH), jnp.float32),
        pltpu.VMEM((CHUNK, DH), jnp.float32),
    ]
    if compute_deg:
        scratch.append(pltpu.VMEM((CHUNK,), jnp.float32))
    scratch.append(pltpu.VMEM_SHARED((NPAD, DH), jnp.float32))
    scratch.append(pltpu.VMEM_SHARED((NPAD, DH), jnp.float32))
    if compute_deg:
        scratch.append(pltpu.VMEM_SHARED((NPAD,), jnp.float32))
    scratch += [
        pltpu.SemaphoreType.DMA((NBUF,)),
        pltpu.SemaphoreType.DMA((NBUF,)),
    ]
    if compute_deg:
        scratch.append(pltpu.SemaphoreType.DMA)
    return pl.kernel(
        _make_seg_body(compute_deg),
        out_type=out_type,
        scratch_types=scratch,
        mesh=plsc.VectorSubcoreMesh(core_axis_name="c", subcore_axis_name="s"),
        compiler_params=pltpu.CompilerParams(use_tc_tiling_on_sc=False,
                                             needs_layout_passes=False),
    )


_seg_sum_deg = _make_seg(True)
_seg_sum = _make_seg(False)


def _dense_body(relu, split_out, s_ref, degt_ref, x_ref, wl_ref, b_ref,
                wr_ref, *outs):
    deg = degt_ref[:, 0:1] + degt_ref[:, 1:2]
    inv = 1.0 / jnp.maximum(deg, 1.0)
    agg = jnp.concatenate(
        [s_ref[0, 0] + s_ref[1, 0], s_ref[0, 1] + s_ref[1, 1]],
        axis=1) * inv
    y = (jnp.dot(agg, wl_ref[...], preferred_element_type=jnp.float32)
         + b_ref[...]
         + jnp.dot(x_ref[...], wr_ref[...], preferred_element_type=jnp.float32))
    if relu:
        y = jnp.maximum(y, 0.0)
    outs[0][...] = y
    if split_out:
        outs[1][0] = y[:, :DH]
        outs[1][1] = y[:, DH:]


def _dense(s, degt, x, w_l, b, w_r, relu, split_out, bm, rows):
    grid = (rows // bm,)
    out_shape = [jax.ShapeDtypeStruct((rows, D), jnp.float32)]
    out_specs = [pl.BlockSpec((bm, D), lambda i: (i, 0))]
    if split_out:
        out_shape.append(jax.ShapeDtypeStruct((2, rows, DH), jnp.float32))
        out_specs.append(pl.BlockSpec((2, bm, DH), lambda i: (0, i, 0)))
    return pl.pallas_call(
        functools.partial(_dense_body, relu, split_out),
        grid=grid,
        in_specs=[
            pl.BlockSpec((NUM_CORES, 2, bm, DH), lambda i: (0, 0, i, 0)),
            pl.BlockSpec((bm, NUM_CORES), lambda i: (i, 0)),
            pl.BlockSpec((bm, D), lambda i: (i, 0)),
            pl.BlockSpec((D, D), lambda i: (0, 0)),
            pl.BlockSpec((1, D), lambda i: (0, 0)),
            pl.BlockSpec((D, D), lambda i: (0, 0)),
        ],
        out_specs=out_specs,
        out_shape=out_shape,
        compiler_params=pltpu.CompilerParams(
            dimension_semantics=("arbitrary",)),
    )(s, degt, x, w_l, b.reshape(1, D), w_r)


def kernel(x, edge_index, W1_l, b1, W1_r, W2_l, b2, W2_r):
    src = edge_index[0].astype(jnp.int32)
    dst = edge_index[1].astype(jnp.int32)
    src = jnp.concatenate([src, jnp.zeros((EPAD - N_EDGES,), jnp.int32)])
    dst = jnp.concatenate([dst, jnp.full((EPAD - N_EDGES,), DUMMY_DST, jnp.int32)])
    src2d = src.reshape(NUM_TILES * N_CHUNKS, CHUNK)
    dst2d = dst.reshape(NUM_TILES * N_CHUNKS, CHUNK)
    x_pad = jnp.pad(x, ((0, NPAD - N_NODES), (0, 0)))
    x_split = jnp.stack([x_pad[:, :DH], x_pad[:, DH:]])

    z2d = jnp.zeros((ROWS_PER_TILE, DH), jnp.float32)
    z1d = jnp.zeros((ROWS_PER_TILE,), jnp.float32)
    ones = jnp.ones((CHUNK,), jnp.float32)

    s1, degp = _seg_sum_deg(x_split, src2d, dst2d, z2d, z1d, ones)
    degt = degp.T
    h, h_split = _dense(s1, degt, x_pad, W1_l, b1, W1_r,
                        relu=True, split_out=True, bm=1264, rows=NPAD)
    (s2,) = _seg_sum(h_split, src2d, dst2d, z2d)
    out, = _dense(s2, degt, h, W2_l, b2, W2_r,
                  relu=False, split_out=False, bm=2000, rows=N_NODES)
    return out


# speedup vs baseline: 10.1406x; 1.0893x over previous
import functools

import jax
import jax.numpy as jnp
from jax import lax
from jax.experimental import pallas as pl
from jax.experimental.pallas import tpu as pltpu
from jax.experimental.pallas import tpu_sc as plsc

N_NODES = 10000
D = 128
DH = D // 2
N_EDGES = 320000

NUM_CORES = 2
NUM_SUBCORES = 16
NUM_TILES = NUM_CORES * NUM_SUBCORES

NPAD = 10112
ROWS_PER_TILE = NPAD // NUM_SUBCORES
DUMMY_DST = N_NODES

EPAD = 327680
E_PER_TILE = EPAD // NUM_TILES
CHUNK = 128
N_CHUNKS = E_PER_TILE // CHUNK
NBUF = 3
N_STEPS = N_CHUNKS + 1
N_ITERS = N_STEPS // NBUF


def _make_seg_body(compute_deg, in_rows):
    def body(*refs):
        if compute_deg:
            (x_hbm, src_hbm, dst_hbm, z2d_hbm, z1d_hbm, ones_hbm,
             s_out, deg_out,
             src_idx, dst_idx, rv0, rv1, rv2, ones_v,
             xsp_sh, acc_sh, deg_sh, gsem, ssem, dsem) = refs
        else:
            (x_hbm, src_hbm, dst_hbm, z2d_hbm, s_out,
             src_idx, dst_idx, rv0, rv1, rv2,
             xsp_sh, acc_sh, gsem, ssem) = refs
        rv = (rv0, rv1, rv2)
        c = lax.axis_index("c")
        s = lax.axis_index("s")
        tid = c * NUM_SUBCORES + s
        rbase = s * ROWS_PER_TILE

        pltpu.sync_copy(src_hbm.at[pl.ds(tid * N_CHUNKS, N_CHUNKS)], src_idx)
        pltpu.sync_copy(dst_hbm.at[pl.ds(tid * N_CHUNKS, N_CHUNKS)], dst_idx)
        if compute_deg:
            pltpu.sync_copy(z1d_hbm, deg_sh.at[pl.ds(rbase, ROWS_PER_TILE)])
            pltpu.sync_copy(ones_hbm, ones_v)

        last_rows = in_rows - (NUM_SUBCORES - 1) * ROWS_PER_TILE

        for p in range(2):
            deg_pass = compute_deg and p == 0
            if last_rows == ROWS_PER_TILE:
                pltpu.sync_copy(
                    x_hbm.at[pl.ds(rbase, ROWS_PER_TILE), pl.ds(p * DH, DH)],
                    xsp_sh.at[pl.ds(rbase, ROWS_PER_TILE)])
            else:
                @pl.when(s < NUM_SUBCORES - 1)
                def _():
                    pltpu.sync_copy(
                        x_hbm.at[pl.ds(rbase, ROWS_PER_TILE),
                                 pl.ds(p * DH, DH)],
                        xsp_sh.at[pl.ds(rbase, ROWS_PER_TILE)])
                @pl.when(s == NUM_SUBCORES - 1)
                def _():
                    lb = (NUM_SUBCORES - 1) * ROWS_PER_TILE
                    pltpu.sync_copy(
                        x_hbm.at[pl.ds(lb, last_rows), pl.ds(p * DH, DH)],
                        xsp_sh.at[pl.ds(lb, last_rows)])
            pltpu.sync_copy(z2d_hbm, acc_sh.at[pl.ds(rbase, ROWS_PER_TILE)])
            plsc.subcore_barrier()

            def step(k, u):
                j = k * NBUF + u
                b = rv[u]
                up = (u - 1) % NBUF

                @pl.when(k > 0)
                def _():
                    pltpu.make_async_copy(b, acc_sh.at[dst_idx.at[0]],
                                          ssem.at[u]).wait()
                @pl.when(j < N_CHUNKS)
                def _():
                    pltpu.async_copy(xsp_sh.at[src_idx.at[j]], b, gsem.at[u])

                i = j - 1
                bp = rv[up]
                @pl.when(i >= 0)
                def _():
                    pltpu.make_async_copy(xsp_sh.at[src_idx.at[i]], bp,
                                          gsem.at[up]).wait()
                    pltpu.async_copy(bp, acc_sh.at[dst_idx.at[i]],
                                     ssem.at[up], add=True)
                    if deg_pass:
                        pltpu.async_copy(ones_v, deg_sh.at[dst_idx.at[i]],
                                         dsem, add=True)

            def it(k, carry):
                for u in range(NBUF):
                    step(k, u)
                return carry

            lax.fori_loop(0, N_ITERS, it, 0)
            for u in range(NBUF):
                if sum(1 for i in range(N_CHUNKS) if i % NBUF == u) > N_ITERS - 1:
                    pltpu.make_async_copy(rv[u], acc_sh.at[dst_idx.at[0]],
                                          ssem.at[u]).wait()
            if deg_pass:
                def dwait(i, carry):
                    pltpu.make_async_copy(ones_v, deg_sh.at[dst_idx.at[0]],
                                          dsem).wait()
                    return carry
                lax.fori_loop(0, N_CHUNKS, dwait, 0)
            plsc.subcore_barrier()

            pltpu.sync_copy(acc_sh.at[pl.ds(rbase, ROWS_PER_TILE)],
                            s_out.at[c, p, pl.ds(rbase, ROWS_PER_TILE)])
            if deg_pass:
                pltpu.sync_copy(deg_sh.at[pl.ds(rbase, ROWS_PER_TILE)],
                                deg_out.at[c, pl.ds(rbase, ROWS_PER_TILE)])
            plsc.subcore_barrier()

    return body


def _make_seg(compute_deg, in_rows):
    out_type = [jax.ShapeDtypeStruct((NUM_CORES, 2, NPAD, DH), jnp.float32)]
    if compute_deg:
        out_type.append(jax.ShapeDtypeStruct((NUM_CORES, NPAD), jnp.float32))
    scratch = [
        pltpu.VMEM((N_CHUNKS, CHUNK), jnp.int32),
        pltpu.VMEM((N_CHUNKS, CHUNK), jnp.int32),
        pltpu.VMEM((CHUNK, DH), jnp.float32),
        pltpu.VMEM((CHUNK, DH), jnp.float32),
        pltpu.VMEM((CHUNK, DH), jnp.float32),
    ]
    if compute_deg:
        scratch.append(pltpu.VMEM((CHUNK,), jnp.float32))
    scratch.append(pltpu.VMEM_SHARED((NPAD, DH), jnp.float32))
    scratch.append(pltpu.VMEM_SHARED((NPAD, DH), jnp.float32))
    if compute_deg:
        scratch.append(pltpu.VMEM_SHARED((NPAD,), jnp.float32))
    scratch += [
        pltpu.SemaphoreType.DMA((NBUF,)),
        pltpu.SemaphoreType.DMA((NBUF,)),
    ]
    if compute_deg:
        scratch.append(pltpu.SemaphoreType.DMA)
    return pl.kernel(
        _make_seg_body(compute_deg, in_rows),
        out_type=out_type,
        scratch_types=scratch,
        mesh=plsc.VectorSubcoreMesh(core_axis_name="c", subcore_axis_name="s"),
        compiler_params=pltpu.CompilerParams(use_tc_tiling_on_sc=False,
                                             needs_layout_passes=False),
    )


_seg_sum_deg = _make_seg(True, N_NODES)
_seg_sum = _make_seg(False, N_NODES)


def _dense_body(relu, s_ref, degt_ref, x_ref, wl_ref, b_ref, wr_ref, o_ref):
    deg = degt_ref[:, 0:1] + degt_ref[:, 1:2]
    inv = 1.0 / jnp.maximum(deg, 1.0)
    agg = jnp.concatenate(
        [s_ref[0, 0] + s_ref[1, 0], s_ref[0, 1] + s_ref[1, 1]],
        axis=1) * inv
    y = (jnp.dot(agg, wl_ref[...], preferred_element_type=jnp.float32)
         + b_ref[...]
         + jnp.dot(x_ref[...], wr_ref[...], preferred_element_type=jnp.float32))
    o_ref[...] = jnp.maximum(y, 0.0) if relu else y


def _dense(s, degt, x, w_l, b, w_r, relu, bm, rows):
    grid = (rows // bm,)
    out_shape = jax.ShapeDtypeStruct((rows, D), jnp.float32)
    out_specs = pl.BlockSpec((bm, D), lambda i: (i, 0))
    return pl.pallas_call(
        functools.partial(_dense_body, relu),
        grid=grid,
        in_specs=[
            pl.BlockSpec((NUM_CORES, 2, bm, DH), lambda i: (0, 0, i, 0)),
            pl.BlockSpec((bm, NUM_CORES), lambda i: (i, 0)),
            pl.BlockSpec((bm, D), lambda i: (i, 0)),
            pl.BlockSpec((D, D), lambda i: (0, 0)),
            pl.BlockSpec((1, D), lambda i: (0, 0)),
            pl.BlockSpec((D, D), lambda i: (0, 0)),
        ],
        out_specs=out_specs,
        out_shape=out_shape,
        compiler_params=pltpu.CompilerParams(
            dimension_semantics=("arbitrary",)),
    )(s, degt, x, w_l, b.reshape(1, D), w_r)


def kernel(x, edge_index, W1_l, b1, W1_r, W2_l, b2, W2_r):
    src = edge_index[0].astype(jnp.int32)
    dst = edge_index[1].astype(jnp.int32)
    src = jnp.concatenate([src, jnp.zeros((EPAD - N_EDGES,), jnp.int32)])
    dst = jnp.concatenate([dst, jnp.full((EPAD - N_EDGES,), DUMMY_DST, jnp.int32)])
    src2d = src.reshape(NUM_TILES * N_CHUNKS, CHUNK)
    dst2d = dst.reshape(NUM_TILES * N_CHUNKS, CHUNK)

    z2d = jnp.zeros((ROWS_PER_TILE, DH), jnp.float32)
    z1d = jnp.zeros((ROWS_PER_TILE,), jnp.float32)
    ones = jnp.ones((CHUNK,), jnp.float32)

    s1, degp = _seg_sum_deg(x, src2d, dst2d, z2d, z1d, ones)
    degt = degp.T
    h = _dense(s1, degt, x, W1_l, b1, W1_r, relu=True, bm=2000, rows=N_NODES)
    (s2,) = _seg_sum(h, src2d, dst2d, z2d)
    return _dense(s2, degt, h, W2_l, b2, W2_r, relu=False, bm=2000,
                  rows=N_NODES)


# speedup vs baseline: 10.1414x; 1.0001x over previous
import functools

import jax
import jax.numpy as jnp
from jax import lax
from jax.experimental import pallas as pl
from jax.experimental.pallas import tpu as pltpu
from jax.experimental.pallas import tpu_sc as plsc

N_NODES = 10000
D = 128
DH = D // 2
N_EDGES = 320000

NUM_CORES = 2
NUM_SUBCORES = 16
NUM_TILES = NUM_CORES * NUM_SUBCORES

NPAD = 10112
ROWS_PER_TILE = NPAD // NUM_SUBCORES
DUMMY_DST = N_NODES

EPAD = 327680
E_PER_TILE = EPAD // NUM_TILES
CHUNK = 128
N_CHUNKS = E_PER_TILE // CHUNK
NBUF = 3
N_STEPS = N_CHUNKS + 1
N_ITERS = N_STEPS // NBUF


def _make_seg_body(compute_deg, in_rows):
    def body(*refs):
        if compute_deg:
            (x_hbm, src_hbm, dst_hbm, z2d_hbm, z1d_hbm, ones_hbm,
             s_out, deg_out,
             src_idx, dst_idx, rv0, rv1, rv2, ones_v,
             xsp_sh, acc_sh, deg_sh, gsem, ssem, dsem) = refs
        else:
            (x_hbm, src_hbm, dst_hbm, z2d_hbm, s_out,
             src_idx, dst_idx, rv0, rv1, rv2,
             xsp_sh, acc_sh, gsem, ssem) = refs
        rv = (rv0, rv1, rv2)
        c = lax.axis_index("c")
        s = lax.axis_index("s")
        tid = c * NUM_SUBCORES + s
        rbase = s * ROWS_PER_TILE

        pltpu.sync_copy(src_hbm.at[pl.ds(tid * N_CHUNKS, N_CHUNKS)], src_idx)
        pltpu.sync_copy(dst_hbm.at[pl.ds(tid * N_CHUNKS, N_CHUNKS)], dst_idx)
        if compute_deg:
            pltpu.sync_copy(z1d_hbm, deg_sh.at[pl.ds(rbase, ROWS_PER_TILE)])
            pltpu.sync_copy(ones_hbm, ones_v)

        last_rows = in_rows - (NUM_SUBCORES - 1) * ROWS_PER_TILE

        for p in range(2):
            deg_pass = compute_deg and p == 0
            if last_rows == ROWS_PER_TILE:
                pltpu.sync_copy(
                    x_hbm.at[pl.ds(rbase, ROWS_PER_TILE), pl.ds(p * DH, DH)],
                    xsp_sh.at[pl.ds(rbase, ROWS_PER_TILE)])
            else:
                @pl.when(s < NUM_SUBCORES - 1)
                def _():
                    pltpu.sync_copy(
                        x_hbm.at[pl.ds(rbase, ROWS_PER_TILE),
                                 pl.ds(p * DH, DH)],
                        xsp_sh.at[pl.ds(rbase, ROWS_PER_TILE)])
                @pl.when(s == NUM_SUBCORES - 1)
                def _():
                    lb = (NUM_SUBCORES - 1) * ROWS_PER_TILE
                    pltpu.sync_copy(
                        x_hbm.at[pl.ds(lb, last_rows), pl.ds(p * DH, DH)],
                        xsp_sh.at[pl.ds(lb, last_rows)])
            pltpu.sync_copy(z2d_hbm, acc_sh.at[pl.ds(rbase, ROWS_PER_TILE)])
            plsc.subcore_barrier()

            def step(k, u):
                j = k * NBUF + u
                b = rv[u]
                up = (u - 1) % NBUF

                @pl.when(k > 0)
                def _():
                    pltpu.make_async_copy(b, acc_sh.at[dst_idx.at[0]],
                                          ssem.at[u]).wait()
                @pl.when(j < N_CHUNKS)
                def _():
                    pltpu.async_copy(xsp_sh.at[src_idx.at[j]], b, gsem.at[u])

                i = j - 1
                bp = rv[up]
                @pl.when(i >= 0)
                def _():
                    pltpu.make_async_copy(xsp_sh.at[src_idx.at[i]], bp,
                                          gsem.at[up]).wait()
                    pltpu.async_copy(bp, acc_sh.at[dst_idx.at[i]],
                                     ssem.at[up], add=True)
                    if deg_pass:
                        pltpu.async_copy(ones_v, deg_sh.at[dst_idx.at[i]],
                                         dsem, add=True)

            def it(k, carry):
                for u in range(NBUF):
                    step(k, u)
                return carry

            lax.fori_loop(0, N_ITERS, it, 0)
            for u in range(NBUF):
                if sum(1 for i in range(N_CHUNKS) if i % NBUF == u) > N_ITERS - 1:
                    pltpu.make_async_copy(rv[u], acc_sh.at[dst_idx.at[0]],
                                          ssem.at[u]).wait()
            if compute_deg and p == 1:
                def dwait(i, carry):
                    pltpu.make_async_copy(ones_v, deg_sh.at[dst_idx.at[0]],
                                          dsem).wait()
                    return carry
                lax.fori_loop(0, N_CHUNKS, dwait, 0)
            plsc.subcore_barrier()

            pltpu.sync_copy(acc_sh.at[pl.ds(rbase, ROWS_PER_TILE)],
                            s_out.at[c, p, pl.ds(rbase, ROWS_PER_TILE)])
            if compute_deg and p == 1:
                pltpu.sync_copy(deg_sh.at[pl.ds(rbase, ROWS_PER_TILE)],
                                deg_out.at[c, pl.ds(rbase, ROWS_PER_TILE)])
            plsc.subcore_barrier()

    return body


def _make_seg(compute_deg, in_rows):
    out_type = [jax.ShapeDtypeStruct((NUM_CORES, 2, NPAD, DH), jnp.float32)]
    if compute_deg:
        out_type.append(jax.ShapeDtypeStruct((NUM_CORES, NPAD), jnp.float32))
    scratch = [
        pltpu.VMEM((N_CHUNKS, CHUNK), jnp.int32),
        pltpu.VMEM((N_CHUNKS, CHUNK), jnp.int32),
        pltpu.VMEM((CHUNK, DH), jnp.float32),
        pltpu.VMEM((CHUNK, DH), jnp.float32),
        pltpu.VMEM((CHUNK, DH), jnp.float32),
    ]
    if compute_deg:
        scratch.append(pltpu.VMEM((CHUNK,), jnp.float32))
    scratch.append(pltpu.VMEM_SHARED((NPAD, DH), jnp.float32))
    scratch.append(pltpu.VMEM_SHARED((NPAD, DH), jnp.float32))
    if compute_deg:
        scratch.append(pltpu.VMEM_SHARED((NPAD,), jnp.float32))
    scratch += [
        pltpu.SemaphoreType.DMA((NBUF,)),
        pltpu.SemaphoreType.DMA((NBUF,)),
    ]
    if compute_deg:
        scratch.append(pltpu.SemaphoreType.DMA)
    return pl.kernel(
        _make_seg_body(compute_deg, in_rows),
        out_type=out_type,
        scratch_types=scratch,
        mesh=plsc.VectorSubcoreMesh(core_axis_name="c", subcore_axis_name="s"),
        compiler_params=pltpu.CompilerParams(use_tc_tiling_on_sc=False,
                                             needs_layout_passes=False),
    )


_seg_sum_deg = _make_seg(True, N_NODES)
_seg_sum = _make_seg(False, N_NODES)


def _dense_body(relu, s_ref, degt_ref, x_ref, wl_ref, b_ref, wr_ref, o_ref):
    deg = degt_ref[:, 0:1] + degt_ref[:, 1:2]
    inv = 1.0 / jnp.maximum(deg, 1.0)
    agg = jnp.concatenate(
        [s_ref[0, 0] + s_ref[1, 0], s_ref[0, 1] + s_ref[1, 1]],
        axis=1) * inv
    y = (jnp.dot(agg, wl_ref[...], preferred_element_type=jnp.float32)
         + b_ref[...]
         + jnp.dot(x_ref[...], wr_ref[...], preferred_element_type=jnp.float32))
    o_ref[...] = jnp.maximum(y, 0.0) if relu else y


def _dense(s, degt, x, w_l, b, w_r, relu, bm, rows):
    grid = (rows // bm,)
    out_shape = jax.ShapeDtypeStruct((rows, D), jnp.float32)
    out_specs = pl.BlockSpec((bm, D), lambda i: (i, 0))
    return pl.pallas_call(
        functools.partial(_dense_body, relu),
        grid=grid,
        in_specs=[
            pl.BlockSpec((NUM_CORES, 2, bm, DH), lambda i: (0, 0, i, 0)),
            pl.BlockSpec((bm, NUM_CORES), lambda i: (i, 0)),
            pl.BlockSpec((bm, D), lambda i: (i, 0)),
            pl.BlockSpec((D, D), lambda i: (0, 0)),
            pl.BlockSpec((1, D), lambda i: (0, 0)),
            pl.BlockSpec((D, D), lambda i: (0, 0)),
        ],
        out_specs=out_specs,
        out_shape=out_shape,
        compiler_params=pltpu.CompilerParams(
            dimension_semantics=("arbitrary",)),
    )(s, degt, x, w_l, b.reshape(1, D), w_r)


def kernel(x, edge_index, W1_l, b1, W1_r, W2_l, b2, W2_r):
    src = edge_index[0].astype(jnp.int32)
    dst = edge_index[1].astype(jnp.int32)
    src = jnp.concatenate([src, jnp.zeros((EPAD - N_EDGES,), jnp.int32)])
    dst = jnp.concatenate([dst, jnp.full((EPAD - N_EDGES,), DUMMY_DST, jnp.int32)])
    src2d = src.reshape(NUM_TILES * N_CHUNKS, CHUNK)
    dst2d = dst.reshape(NUM_TILES * N_CHUNKS, CHUNK)

    z2d = jnp.zeros((ROWS_PER_TILE, DH), jnp.float32)
    z1d = jnp.zeros((ROWS_PER_TILE,), jnp.float32)
    ones = jnp.ones((CHUNK,), jnp.float32)

    s1, degp = _seg_sum_deg(x, src2d, dst2d, z2d, z1d, ones)
    degt = degp.T
    h = _dense(s1, degt, x, W1_l, b1, W1_r, relu=True, bm=2000, rows=N_NODES)
    (s2,) = _seg_sum(h, src2d, dst2d, z2d)
    return _dense(s2, degt, h, W2_l, b2, W2_r, relu=False, bm=2000,
                  rows=N_NODES)
